# baseline (speedup 1.0000x reference)
import functools

import jax
import jax.numpy as jnp
from jax import lax
from jax.experimental import pallas as pl
from jax.experimental.pallas import tpu as pltpu
from jax.experimental.pallas import tpu_sc as plsc

SEQ = 200
EMBED = 64
HID = 128
GATES = 4 * HID

_NC = 2
_NS = 16
_B_PER_W = 8
_NW_USED = SEQ // _B_PER_W


def _sc_gather_body(table_hbm, idx_hbm, out_hbm, idx_v, rows_v, sem):
    wid = lax.axis_index("s") * _NC + lax.axis_index("c")

    @pl.when(wid < _NW_USED)
    def _():
        base = wid * _B_PER_W
        pltpu.sync_copy(idx_hbm.at[pl.ds(base, _B_PER_W)], idx_v)
        pltpu.async_copy(table_hbm.at[idx_v], rows_v, sem).wait()
        pltpu.sync_copy(rows_v, out_hbm.at[pl.ds(base, _B_PER_W)])


def _sc_gather(table, idx):
    mesh = plsc.VectorSubcoreMesh(core_axis_name="c", subcore_axis_name="s")
    f = functools.partial(
        pl.kernel,
        mesh=mesh,
        out_type=jax.ShapeDtypeStruct((SEQ, EMBED), jnp.float32),
        scratch_types=[
            pltpu.VMEM((_B_PER_W,), jnp.int32),
            pltpu.VMEM((_B_PER_W, EMBED), jnp.float32),
            pltpu.SemaphoreType.DMA,
        ],
        compiler_params=pltpu.CompilerParams(use_tc_tiling_on_sc=False),
    )(_sc_gather_body)
    return f(table, idx)


def _lstm_body(e_ref, wih_ref, whh_ref, bih_ref, bhh_ref, wout_ref, bout_ref,
               x_ref, gx_ref):
    b = (bih_ref[...] + bhh_ref[...]).reshape(1, GATES)
    gx_ref[...] = (
        lax.dot_general(
            e_ref[...], wih_ref[...],
            (((1,), (1,)), ((), ())),
            preferred_element_type=jnp.float32,
        )
        + b
    )
    whh_f = whh_ref[...]
    w_hi = whh_f.astype(jnp.bfloat16)
    w_lo = (whh_f - w_hi.astype(jnp.float32)).astype(jnp.bfloat16)

    def step(t, carry):
        h, c = carry
        hb = h.astype(jnp.bfloat16)
        g = (
            gx_ref[pl.ds(t, 1), :]
            + lax.dot_general(hb, w_hi, (((1,), (1,)), ((), ())),
                              preferred_element_type=jnp.float32)
            + lax.dot_general(hb, w_lo, (((1,), (1,)), ((), ())),
                              preferred_element_type=jnp.float32)
        )
        i = jax.nn.sigmoid(g[:, 0:HID])
        f = jax.nn.sigmoid(g[:, HID:2 * HID])
        gg = jnp.tanh(g[:, 2 * HID:3 * HID])
        o = jax.nn.sigmoid(g[:, 3 * HID:4 * HID])
        c = f * c + i * gg
        h = o * jnp.tanh(c)
        return (h, c)

    h0 = jnp.zeros((1, HID), jnp.float32)
    c0 = jnp.zeros((1, HID), jnp.float32)
    h, _ = lax.fori_loop(0, SEQ, step, (h0, c0), unroll=2)
    x_ref[...] = (
        lax.dot_general(
            h, wout_ref[...], (((1,), (1,)), ((), ())),
            preferred_element_type=jnp.float32,
        )
        + bout_ref[...][None, :]
    )


def _lstm(rows, W_ih, W_hh, b_ih, b_hh, W_out, b_out):
    return pl.pallas_call(
        _lstm_body,
        out_shape=jax.ShapeDtypeStruct((1, 2), jnp.float32),
        in_specs=[
            pl.BlockSpec(memory_space=pltpu.VMEM),
            pl.BlockSpec(memory_space=pltpu.VMEM),
            pl.BlockSpec(memory_space=pltpu.VMEM),
            pl.BlockSpec(memory_space=pltpu.VMEM),
            pl.BlockSpec(memory_space=pltpu.VMEM),
            pl.BlockSpec(memory_space=pltpu.VMEM),
            pl.BlockSpec(memory_space=pltpu.VMEM),
        ],
        scratch_shapes=[
            pltpu.VMEM((SEQ, GATES), jnp.float32),
        ],
    )(rows, W_ih, W_hh, b_ih, b_hh, W_out, b_out)


def kernel(inputs, emb, W_ih, W_hh, b_ih, b_hh, W_out, b_out):
    idx = inputs.astype(jnp.int32)
    embeddings = _sc_gather(emb, idx)
    x = _lstm(embeddings, W_ih, W_hh, b_ih, b_hh, W_out, b_out)
    return (x, embeddings.reshape(SEQ, 1, EMBED))


# speedup vs baseline: 1.0521x; 1.0521x over previous
import functools

import jax
import jax.numpy as jnp
from jax import lax
from jax.experimental import pallas as pl
from jax.experimental.pallas import tpu as pltpu
from jax.experimental.pallas import tpu_sc as plsc

SEQ = 200
EMBED = 64
HID = 128
GATES = 4 * HID

_NC = 2
_NS = 16
_B_PER_W = 8
_NW_USED = SEQ // _B_PER_W


def _sc_gather_body(table_hbm, idx_hbm, out_hbm, idx_v, rows_v, sem):
    wid = lax.axis_index("s") * _NC + lax.axis_index("c")

    @pl.when(wid < _NW_USED)
    def _():
        base = wid * _B_PER_W
        pltpu.sync_copy(idx_hbm.at[pl.ds(base, _B_PER_W)], idx_v)
        pltpu.async_copy(table_hbm.at[idx_v], rows_v, sem).wait()
        pltpu.sync_copy(rows_v, out_hbm.at[pl.ds(base, _B_PER_W)])


def _sc_gather(table, idx):
    mesh = plsc.VectorSubcoreMesh(core_axis_name="c", subcore_axis_name="s")
    f = functools.partial(
        pl.kernel,
        mesh=mesh,
        out_type=jax.ShapeDtypeStruct((SEQ, EMBED), jnp.float32),
        scratch_types=[
            pltpu.VMEM((_B_PER_W,), jnp.int32),
            pltpu.VMEM((_B_PER_W, EMBED), jnp.float32),
            pltpu.SemaphoreType.DMA,
        ],
        compiler_params=pltpu.CompilerParams(use_tc_tiling_on_sc=False),
    )(_sc_gather_body)
    return f(table, idx)


def _lstm_body(e_ref, wih_ref, whh_ref, bih_ref, bhh_ref, wout_ref, bout_ref,
               x_ref, gx_ref):
    b = (bih_ref[...] + bhh_ref[...]).reshape(1, GATES)
    gx_ref[...] = (
        lax.dot_general(
            e_ref[...], wih_ref[...],
            (((1,), (1,)), ((), ())),
            preferred_element_type=jnp.float32,
        )
        + b
    )
    whh_b = whh_ref[...].astype(jnp.bfloat16)

    def step(t, carry):
        h, c = carry
        g = gx_ref[pl.ds(t, 1), :] + lax.dot_general(
            h.astype(jnp.bfloat16), whh_b, (((1,), (1,)), ((), ())),
            preferred_element_type=jnp.float32,
        )
        i = jax.nn.sigmoid(g[:, 0:HID])
        f = jax.nn.sigmoid(g[:, HID:2 * HID])
        gg = jnp.tanh(g[:, 2 * HID:3 * HID])
        o = jax.nn.sigmoid(g[:, 3 * HID:4 * HID])
        c = f * c + i * gg
        h = o * jnp.tanh(c)
        return (h, c)

    h0 = jnp.zeros((1, HID), jnp.float32)
    c0 = jnp.zeros((1, HID), jnp.float32)
    h, _ = lax.fori_loop(0, SEQ, step, (h0, c0), unroll=2)
    x_ref[...] = (
        lax.dot_general(
            h, wout_ref[...], (((1,), (1,)), ((), ())),
            preferred_element_type=jnp.float32,
        )
        + bout_ref[...][None, :]
    )


def _lstm(rows, W_ih, W_hh, b_ih, b_hh, W_out, b_out):
    return pl.pallas_call(
        _lstm_body,
        out_shape=jax.ShapeDtypeStruct((1, 2), jnp.float32),
        in_specs=[
            pl.BlockSpec(memory_space=pltpu.VMEM),
            pl.BlockSpec(memory_space=pltpu.VMEM),
            pl.BlockSpec(memory_space=pltpu.VMEM),
            pl.BlockSpec(memory_space=pltpu.VMEM),
            pl.BlockSpec(memory_space=pltpu.VMEM),
            pl.BlockSpec(memory_space=pltpu.VMEM),
            pl.BlockSpec(memory_space=pltpu.VMEM),
        ],
        scratch_shapes=[
            pltpu.VMEM((SEQ, GATES), jnp.float32),
        ],
    )(rows, W_ih, W_hh, b_ih, b_hh, W_out, b_out)


def kernel(inputs, emb, W_ih, W_hh, b_ih, b_hh, W_out, b_out):
    idx = inputs.astype(jnp.int32)
    embeddings = _sc_gather(emb, idx)
    x = _lstm(embeddings, W_ih, W_hh, b_ih, b_hh, W_out, b_out)
    return (x, embeddings.reshape(SEQ, 1, EMBED))


# speedup vs baseline: 1.0717x; 1.0186x over previous
import functools

import jax
import jax.numpy as jnp
from jax import lax
from jax.experimental import pallas as pl
from jax.experimental.pallas import tpu as pltpu
from jax.experimental.pallas import tpu_sc as plsc

SEQ = 200
EMBED = 64
HID = 128
GATES = 4 * HID

_NC = 2
_NS = 16
_B_PER_W = 8
_NW_USED = SEQ // _B_PER_W


def _sc_gather_body(table_hbm, idx_hbm, out_hbm, idx_v, rows_v, sem):
    wid = lax.axis_index("s") * _NC + lax.axis_index("c")

    @pl.when(wid < _NW_USED)
    def _():
        base = wid * _B_PER_W
        pltpu.sync_copy(idx_hbm.at[pl.ds(base, _B_PER_W)], idx_v)
        pltpu.async_copy(table_hbm.at[idx_v], rows_v, sem).wait()
        pltpu.sync_copy(rows_v, out_hbm.at[pl.ds(base, _B_PER_W), 0])


def _sc_gather(table, idx):
    mesh = plsc.VectorSubcoreMesh(core_axis_name="c", subcore_axis_name="s")
    f = functools.partial(
        pl.kernel,
        mesh=mesh,
        out_type=jax.ShapeDtypeStruct((SEQ, 1, EMBED), jnp.float32),
        scratch_types=[
            pltpu.VMEM((_B_PER_W,), jnp.int32),
            pltpu.VMEM((_B_PER_W, EMBED), jnp.float32),
            pltpu.SemaphoreType.DMA,
        ],
        compiler_params=pltpu.CompilerParams(use_tc_tiling_on_sc=False),
    )(_sc_gather_body)
    return f(table, idx)


def _lstm_body(e_ref, wih_ref, whh_ref, bih_ref, bhh_ref, wout_ref, bout_ref,
               x_ref, gx_ref):
    b = (bih_ref[...] + bhh_ref[...]).reshape(1, GATES)
    gx_ref[...] = (
        lax.dot_general(
            e_ref[...].reshape(SEQ, EMBED), wih_ref[...],
            (((1,), (1,)), ((), ())),
            preferred_element_type=jnp.float32,
        )
        + b
    )
    whh = whh_ref[...]

    def step(t, carry):
        h, c = carry
        g = gx_ref[pl.ds(t, 1), :] + lax.dot_general(
            h, whh, (((1,), (1,)), ((), ())), preferred_element_type=jnp.float32
        )
        i = jax.nn.sigmoid(g[:, 0:HID])
        f = jax.nn.sigmoid(g[:, HID:2 * HID])
        gg = jnp.tanh(g[:, 2 * HID:3 * HID])
        o = jax.nn.sigmoid(g[:, 3 * HID:4 * HID])
        c = f * c + i * gg
        h = o * jnp.tanh(c)
        return (h, c)

    h0 = jnp.zeros((1, HID), jnp.float32)
    c0 = jnp.zeros((1, HID), jnp.float32)
    h, _ = lax.fori_loop(0, SEQ, step, (h0, c0), unroll=2)
    x_ref[...] = (
        lax.dot_general(
            h, wout_ref[...], (((1,), (1,)), ((), ())),
            preferred_element_type=jnp.float32,
        )
        + bout_ref[...][None, :]
    )


def _lstm(rows, W_ih, W_hh, b_ih, b_hh, W_out, b_out):
    return pl.pallas_call(
        _lstm_body,
        out_shape=jax.ShapeDtypeStruct((1, 2), jnp.float32),
        in_specs=[
            pl.BlockSpec(memory_space=pltpu.VMEM),
            pl.BlockSpec(memory_space=pltpu.VMEM),
            pl.BlockSpec(memory_space=pltpu.VMEM),
            pl.BlockSpec(memory_space=pltpu.VMEM),
            pl.BlockSpec(memory_space=pltpu.VMEM),
            pl.BlockSpec(memory_space=pltpu.VMEM),
            pl.BlockSpec(memory_space=pltpu.VMEM),
        ],
        scratch_shapes=[
            pltpu.VMEM((SEQ, GATES), jnp.float32),
        ],
    )(rows, W_ih, W_hh, b_ih, b_hh, W_out, b_out)


def kernel(inputs, emb, W_ih, W_hh, b_ih, b_hh, W_out, b_out):
    idx = inputs.astype(jnp.int32)
    embeddings = _sc_gather(emb, idx)
    x = _lstm(embeddings, W_ih, W_hh, b_ih, b_hh, W_out, b_out)
    return (x, embeddings)


# speedup vs baseline: 1.0821x; 1.0098x over previous
import functools

import jax
import jax.numpy as jnp
from jax import lax
from jax.experimental import pallas as pl
from jax.experimental.pallas import tpu as pltpu
from jax.experimental.pallas import tpu_sc as plsc

SEQ = 200
EMBED = 64
HID = 128
GATES = 4 * HID

_NC = 2
_NS = 16
_B_PER_W = 8
_NW_USED = SEQ // _B_PER_W


def _sc_gather_body(table_hbm, idx_hbm, out_hbm, idx_v, rows_v, sem):
    wid = lax.axis_index("s") * _NC + lax.axis_index("c")

    @pl.when(wid < _NW_USED)
    def _():
        base = wid * _B_PER_W
        pltpu.sync_copy(idx_hbm.at[pl.ds(base, _B_PER_W)], idx_v)
        pltpu.async_copy(table_hbm.at[idx_v], rows_v, sem).wait()
        pltpu.sync_copy(rows_v, out_hbm.at[pl.ds(base, _B_PER_W), 0])


def _sc_gather(table, idx):
    mesh = plsc.VectorSubcoreMesh(core_axis_name="c", subcore_axis_name="s")
    f = functools.partial(
        pl.kernel,
        mesh=mesh,
        out_type=jax.ShapeDtypeStruct((SEQ, 1, EMBED), jnp.float32),
        scratch_types=[
            pltpu.VMEM((_B_PER_W,), jnp.int32),
            pltpu.VMEM((_B_PER_W, EMBED), jnp.float32),
            pltpu.SemaphoreType.DMA,
        ],
        compiler_params=pltpu.CompilerParams(use_tc_tiling_on_sc=False),
    )(_sc_gather_body)
    return f(table, idx)


def _lstm_body(e_ref, wih_ref, whh_ref, bih_ref, bhh_ref, wout_ref, bout_ref,
               x_ref, gx_ref):
    b = (bih_ref[...] + bhh_ref[...]).reshape(1, GATES)
    gx_ref[...] = (
        lax.dot_general(
            e_ref[...].reshape(SEQ, EMBED), wih_ref[...],
            (((1,), (1,)), ((), ())),
            preferred_element_type=jnp.float32,
        )
        + b
    )
    whh = whh_ref[...]

    def step(t, carry):
        h, c = carry
        g = gx_ref[pl.ds(t, 1), :] + lax.dot_general(
            h, whh, (((1,), (1,)), ((), ())), preferred_element_type=jnp.float32
        )
        i = jax.nn.sigmoid(g[:, 0:HID])
        f = jax.nn.sigmoid(g[:, HID:2 * HID])
        gg = jnp.tanh(g[:, 2 * HID:3 * HID])
        o = jax.nn.sigmoid(g[:, 3 * HID:4 * HID])
        c = f * c + i * gg
        h = o * jnp.tanh(c)
        return (h, c)

    h0 = jnp.zeros((1, HID), jnp.float32)
    c0 = jnp.zeros((1, HID), jnp.float32)
    h, _ = lax.fori_loop(0, SEQ, step, (h0, c0), unroll=4)
    x_ref[...] = (
        lax.dot_general(
            h, wout_ref[...], (((1,), (1,)), ((), ())),
            preferred_element_type=jnp.float32,
        )
        + bout_ref[...][None, :]
    )


def _lstm(rows, W_ih, W_hh, b_ih, b_hh, W_out, b_out):
    return pl.pallas_call(
        _lstm_body,
        out_shape=jax.ShapeDtypeStruct((1, 2), jnp.float32),
        in_specs=[
            pl.BlockSpec(memory_space=pltpu.VMEM),
            pl.BlockSpec(memory_space=pltpu.VMEM),
            pl.BlockSpec(memory_space=pltpu.VMEM),
            pl.BlockSpec(memory_space=pltpu.VMEM),
            pl.BlockSpec(memory_space=pltpu.VMEM),
            pl.BlockSpec(memory_space=pltpu.VMEM),
            pl.BlockSpec(memory_space=pltpu.VMEM),
        ],
        scratch_shapes=[
            pltpu.VMEM((SEQ, GATES), jnp.float32),
        ],
    )(rows, W_ih, W_hh, b_ih, b_hh, W_out, b_out)


def kernel(inputs, emb, W_ih, W_hh, b_ih, b_hh, W_out, b_out):
    idx = inputs.astype(jnp.int32)
    embeddings = _sc_gather(emb, idx)
    x = _lstm(embeddings, W_ih, W_hh, b_ih, b_hh, W_out, b_out)
    return (x, embeddings)


# speedup vs baseline: 1.0967x; 1.0135x over previous
import functools

import jax
import jax.numpy as jnp
from jax import lax
from jax.experimental import pallas as pl
from jax.experimental.pallas import tpu as pltpu
from jax.experimental.pallas import tpu_sc as plsc

SEQ = 200
EMBED = 64
HID = 128
GATES = 4 * HID

_NC = 2
_NS = 16
_B_PER_W = 8
_NW_USED = SEQ // _B_PER_W


def _sc_gather_body(table_hbm, idx_hbm, out_hbm, idx_v, rows_v, sem):
    wid = lax.axis_index("s") * _NC + lax.axis_index("c")

    @pl.when(wid < _NW_USED)
    def _():
        base = wid * _B_PER_W
        pltpu.sync_copy(idx_hbm.at[pl.ds(base, _B_PER_W)], idx_v)
        pltpu.async_copy(table_hbm.at[idx_v], rows_v, sem).wait()
        pltpu.sync_copy(rows_v, out_hbm.at[pl.ds(base, _B_PER_W), 0])


def _sc_gather(table, idx):
    mesh = plsc.VectorSubcoreMesh(core_axis_name="c", subcore_axis_name="s")
    f = functools.partial(
        pl.kernel,
        mesh=mesh,
        out_type=jax.ShapeDtypeStruct((SEQ, 1, EMBED), jnp.float32),
        scratch_types=[
            pltpu.VMEM((_B_PER_W,), jnp.int32),
            pltpu.VMEM((_B_PER_W, EMBED), jnp.float32),
            pltpu.SemaphoreType.DMA,
        ],
        compiler_params=pltpu.CompilerParams(use_tc_tiling_on_sc=False),
    )(_sc_gather_body)
    return f(table, idx)


def _lstm_body(e_ref, wih_ref, whh_ref, bih_ref, bhh_ref, wout_ref, bout_ref,
               x_ref, gx_ref):
    b = (bih_ref[...] + bhh_ref[...]).reshape(1, GATES)
    gx_ref[...] = (
        lax.dot_general(
            e_ref[...].reshape(SEQ, EMBED), wih_ref[...],
            (((1,), (1,)), ((), ())),
            preferred_element_type=jnp.float32,
        )
        + b
    )
    whh = whh_ref[...]

    def step(t, carry):
        h, c = carry
        g = gx_ref[pl.ds(t, 1), :] + lax.dot_general(
            h, whh, (((1,), (1,)), ((), ())), preferred_element_type=jnp.float32
        )
        i = jax.nn.sigmoid(g[:, 0:HID])
        f = jax.nn.sigmoid(g[:, HID:2 * HID])
        gg = jnp.tanh(g[:, 2 * HID:3 * HID])
        o = jax.nn.sigmoid(g[:, 3 * HID:4 * HID])
        c = f * c + i * gg
        h = o * jnp.tanh(c)
        return (h, c)

    h0 = jnp.zeros((1, HID), jnp.float32)
    c0 = jnp.zeros((1, HID), jnp.float32)
    h, _ = lax.fori_loop(0, SEQ, step, (h0, c0), unroll=8)
    x_ref[...] = (
        lax.dot_general(
            h, wout_ref[...], (((1,), (1,)), ((), ())),
            preferred_element_type=jnp.float32,
        )
        + bout_ref[...][None, :]
    )


def _lstm(rows, W_ih, W_hh, b_ih, b_hh, W_out, b_out):
    return pl.pallas_call(
        _lstm_body,
        out_shape=jax.ShapeDtypeStruct((1, 2), jnp.float32),
        in_specs=[
            pl.BlockSpec(memory_space=pltpu.VMEM),
            pl.BlockSpec(memory_space=pltpu.VMEM),
            pl.BlockSpec(memory_space=pltpu.VMEM),
            pl.BlockSpec(memory_space=pltpu.VMEM),
            pl.BlockSpec(memory_space=pltpu.VMEM),
            pl.BlockSpec(memory_space=pltpu.VMEM),
            pl.BlockSpec(memory_space=pltpu.VMEM),
        ],
        scratch_shapes=[
            pltpu.VMEM((SEQ, GATES), jnp.float32),
        ],
    )(rows, W_ih, W_hh, b_ih, b_hh, W_out, b_out)


def kernel(inputs, emb, W_ih, W_hh, b_ih, b_hh, W_out, b_out):
    idx = inputs.astype(jnp.int32)
    embeddings = _sc_gather(emb, idx)
    x = _lstm(embeddings, W_ih, W_hh, b_ih, b_hh, W_out, b_out)
    return (x, embeddings)


# speedup vs baseline: 1.1016x; 1.0044x over previous
import functools

import jax
import jax.numpy as jnp
from jax import lax
from jax.experimental import pallas as pl
from jax.experimental.pallas import tpu as pltpu
from jax.experimental.pallas import tpu_sc as plsc

SEQ = 200
EMBED = 64
HID = 128
GATES = 4 * HID

_NC = 2
_NS = 16
_B_PER_W = 8
_NW_USED = SEQ // _B_PER_W


def _sc_gather_body(table_hbm, idx_hbm, out_hbm, idx_v, rows_v, sem):
    wid = lax.axis_index("s") * _NC + lax.axis_index("c")

    @pl.when(wid < _NW_USED)
    def _():
        base = wid * _B_PER_W
        pltpu.sync_copy(idx_hbm.at[pl.ds(base, _B_PER_W)], idx_v)
        pltpu.async_copy(table_hbm.at[idx_v], rows_v, sem).wait()
        pltpu.sync_copy(rows_v, out_hbm.at[pl.ds(base, _B_PER_W), 0])


def _sc_gather(table, idx):
    mesh = plsc.VectorSubcoreMesh(core_axis_name="c", subcore_axis_name="s")
    f = functools.partial(
        pl.kernel,
        mesh=mesh,
        out_type=jax.ShapeDtypeStruct((SEQ, 1, EMBED), jnp.float32),
        scratch_types=[
            pltpu.VMEM((_B_PER_W,), jnp.int32),
            pltpu.VMEM((_B_PER_W, EMBED), jnp.float32),
            pltpu.SemaphoreType.DMA,
        ],
        compiler_params=pltpu.CompilerParams(use_tc_tiling_on_sc=False),
    )(_sc_gather_body)
    return f(table, idx)


def _lstm_body(e_ref, wih_ref, whh_ref, bih_ref, bhh_ref, wout_ref, bout_ref,
               x_ref, gx_ref):
    b = (bih_ref[...] + bhh_ref[...]).reshape(1, GATES)
    gx_ref[...] = (
        lax.dot_general(
            e_ref[...].reshape(SEQ, EMBED), wih_ref[...],
            (((1,), (1,)), ((), ())),
            preferred_element_type=jnp.float32,
        )
        + b
    )
    whh = whh_ref[...]

    def step(t, carry):
        h, c = carry
        g = gx_ref[pl.ds(t, 1), :] + lax.dot_general(
            h, whh, (((1,), (1,)), ((), ())), preferred_element_type=jnp.float32
        )
        i = jax.nn.sigmoid(g[:, 0:HID])
        f = jax.nn.sigmoid(g[:, HID:2 * HID])
        gg = jnp.tanh(g[:, 2 * HID:3 * HID])
        o = jax.nn.sigmoid(g[:, 3 * HID:4 * HID])
        c = f * c + i * gg
        h = o * jnp.tanh(c)
        return (h, c)

    h0 = jnp.zeros((1, HID), jnp.float32)
    c0 = jnp.zeros((1, HID), jnp.float32)
    h, _ = lax.fori_loop(0, SEQ, step, (h0, c0), unroll=25)
    x_ref[...] = (
        lax.dot_general(
            h, wout_ref[...], (((1,), (1,)), ((), ())),
            preferred_element_type=jnp.float32,
        )
        + bout_ref[...][None, :]
    )


def _lstm(rows, W_ih, W_hh, b_ih, b_hh, W_out, b_out):
    return pl.pallas_call(
        _lstm_body,
        out_shape=jax.ShapeDtypeStruct((1, 2), jnp.float32),
        in_specs=[
            pl.BlockSpec(memory_space=pltpu.VMEM),
            pl.BlockSpec(memory_space=pltpu.VMEM),
            pl.BlockSpec(memory_space=pltpu.VMEM),
            pl.BlockSpec(memory_space=pltpu.VMEM),
            pl.BlockSpec(memory_space=pltpu.VMEM),
            pl.BlockSpec(memory_space=pltpu.VMEM),
            pl.BlockSpec(memory_space=pltpu.VMEM),
        ],
        scratch_shapes=[
            pltpu.VMEM((SEQ, GATES), jnp.float32),
        ],
    )(rows, W_ih, W_hh, b_ih, b_hh, W_out, b_out)


def kernel(inputs, emb, W_ih, W_hh, b_ih, b_hh, W_out, b_out):
    idx = inputs.astype(jnp.int32)
    embeddings = _sc_gather(emb, idx)
    x = _lstm(embeddings, W_ih, W_hh, b_ih, b_hh, W_out, b_out)
    return (x, embeddings)
